# Initial kernel scaffold; baseline (speedup 1.0000x reference)
import functools

import jax
import jax.numpy as jnp
from jax import lax
from jax.experimental import pallas as pl
from jax.experimental.pallas import tpu as pltpu
from jax.experimental.pallas import tpu_sc as plsc

NC, NS, L = 2, 16, 16
NW = NC * NS


def _prep_body(h_ref, wfc_ref, a2_ref, whaug_ref, alphas_ref):
    wh = lax.dot_general(h_ref[...], wfc_ref[...], (((1,), (1,)), ((), ())),
                         preferred_element_type=jnp.float32)
    alphas_ref[...] = jnp.dot(wh, a2_ref[...], preferred_element_type=jnp.float32)
    whaug_ref[:, : wh.shape[1]] = wh
    br = wh.shape[0]
    lane = lax.broadcasted_iota(jnp.int32, (br, 16), 1)
    whaug_ref[:, wh.shape[1]:] = jnp.where(lane == 0, 1.0, 0.0).astype(jnp.float32)


def _fin_body(p_ref, o_ref, *, d):
    tot = p_ref[0] + p_ref[1]
    num = tot[:, :d]
    s = tot[:, d : d + 1]
    safe = jnp.where(s == 0.0, 1.0, s)
    r = num / safe
    o_ref[...] = jnp.where(r > 0.0, r, jnp.expm1(r))


def _make_sc_kernel(n, n_acc, da, e_pad, k):
    m = e_pad // NW
    n_chunks = m // k
    n_slice = n_acc // NS
    mesh = plsc.VectorSubcoreMesh(core_axis_name="c", subcore_axis_name="s",
                                  num_cores=NC, num_subcores=NS)

    @functools.partial(
        pl.kernel,
        out_type=jax.ShapeDtypeStruct((NC, n_acc, da), jnp.float32),
        mesh=mesh,
        scratch_types=[
            pltpu.VMEM((n,), jnp.float32),
            pltpu.VMEM((n,), jnp.float32),
            pltpu.VMEM((k,), jnp.int32),
            pltpu.VMEM((k,), jnp.int32),
            pltpu.VMEM((k,), jnp.float32),
            pltpu.VMEM((k, da), jnp.float32),
            pltpu.VMEM_SHARED((n_acc, da), jnp.float32),
            pltpu.SemaphoreType.DMA,
        ],
    )
    def sc_kernel(asrc_hbm, adst_hbm, src_hbm, dst_hbm, whaug_hbm, out_hbm,
                  asrc_v, adst_v, src_v, dst_v, ex_v, rows_v, acc, sem):
        cid = lax.axis_index("c")
        sid = lax.axis_index("s")
        wid = sid * NC + cid

        pltpu.sync_copy(asrc_hbm, asrc_v)
        pltpu.sync_copy(adst_hbm, adst_v)

        @pl.loop(0, k)
        def _(r):
            for c in range(da // L):
                rows_v.at[r][pl.ds(c * L, L)] = jnp.zeros((L,), jnp.float32)

        for p in range(n_slice // k):
            pltpu.sync_copy(rows_v, acc.at[pl.ds(sid * n_slice + p * k, k)])
        plsc.subcore_barrier()

        @pl.loop(0, n_chunks)
        def _(ci):
            base = wid * m + ci * k
            pltpu.sync_copy(src_hbm.at[pl.ds(base, k)], src_v)
            pltpu.sync_copy(dst_hbm.at[pl.ds(base, k)], dst_v)
            pltpu.async_copy(whaug_hbm.at[src_v], rows_v, sem).wait()

            @pl.loop(0, k // L)
            def _(g):
                s16 = src_v[pl.ds(g * L, L)]
                d16 = dst_v[pl.ds(g * L, L)]
                e = plsc.load_gather(asrc_v, [s16]) + plsc.load_gather(adst_v, [d16])
                e = jnp.maximum(e, 0.2 * e)
                ex_v[pl.ds(g * L, L)] = jnp.exp(e)

            @pl.loop(0, k)
            def _(r):
                b = plsc.load_gather(ex_v, [jnp.full((L,), r, jnp.int32)])
                for c in range(da // L):
                    rows_v.at[r][pl.ds(c * L, L)] = rows_v.at[r][pl.ds(c * L, L)] * b

            pltpu.sync_copy(rows_v, acc.at[dst_v], add=True)

        plsc.subcore_barrier()
        for p in range(n_slice // k):
            sl = pl.ds(sid * n_slice + p * k, k)
            pltpu.sync_copy(acc.at[sl], out_hbm.at[cid].at[sl])

    return sc_kernel


@jax.jit
def kernel(h, edge_index, W_fc, W_attn):
    b, n, d = h.shape
    e = edge_index.shape[1]
    da = d + 16
    n_acc = ((n + 1 + 1023) // 1024) * 1024
    k = 128
    e_pad = ((e + NW * k - 1) // (NW * k)) * (NW * k)

    h2 = h.reshape(n, d)
    a2 = W_attn.reshape(2, d).T
    src = edge_index[0]
    dst = edge_index[1]
    pad = e_pad - e
    if pad:
        src = jnp.concatenate([src, jnp.zeros((pad,), jnp.int32)])
        dst = jnp.concatenate([dst, jnp.full((pad,), n, jnp.int32)])

    br = 1000
    whaug, alphas = pl.pallas_call(
        _prep_body,
        grid=(n // br,),
        in_specs=[
            pl.BlockSpec((br, d), lambda i: (i, 0)),
            pl.BlockSpec((d, d), lambda i: (0, 0)),
            pl.BlockSpec((d, 2), lambda i: (0, 0)),
        ],
        out_specs=[
            pl.BlockSpec((br, da), lambda i: (i, 0)),
            pl.BlockSpec((br, 2), lambda i: (i, 0)),
        ],
        out_shape=[
            jax.ShapeDtypeStruct((n, da), jnp.float32),
            jax.ShapeDtypeStruct((n, 2), jnp.float32),
        ],
    )(h2, W_fc, a2)

    asrc = jnp.ascontiguousarray(alphas[:, 0])
    adst = jnp.ascontiguousarray(alphas[:, 1])

    parts = _make_sc_kernel(n, n_acc, da, e_pad, k)(asrc, adst, src, dst, whaug)

    bf = 1024
    out = pl.pallas_call(
        functools.partial(_fin_body, d=d),
        grid=(n_acc // bf,),
        in_specs=[pl.BlockSpec((NC, bf, da), lambda i: (0, i, 0))],
        out_specs=pl.BlockSpec((bf, d), lambda i: (i, 0)),
        out_shape=jax.ShapeDtypeStruct((n_acc, d), jnp.float32),
    )(parts)

    return out[:n].reshape(b, n, d)


# speedup vs baseline: 11.2887x; 11.2887x over previous
import dataclasses
import functools

import jax
import jax.numpy as jnp
from jax import lax
from jax.experimental import pallas as pl
from jax.experimental.pallas import tpu as pltpu
from jax.experimental.pallas import tpu_sc as plsc

NC, NS, L = 2, 16, 16
NW = NC * NS


def _prep_body(h_ref, wfc_ref, a2_ref, whaug_ref, alphas_ref):
    wh = lax.dot_general(h_ref[...], wfc_ref[...], (((1,), (1,)), ((), ())),
                         preferred_element_type=jnp.float32)
    alphas_ref[...] = jnp.dot(wh, a2_ref[...], preferred_element_type=jnp.float32)
    whaug_ref[:, : wh.shape[1]] = wh
    br = wh.shape[0]
    lane = lax.broadcasted_iota(jnp.int32, (br, 16), 1)
    whaug_ref[:, wh.shape[1]:] = jnp.where(lane == 0, 1.0, 0.0).astype(jnp.float32)


def _fin_body(p_ref, o_ref, *, d):
    tot = p_ref[0] + p_ref[1]
    num = tot[:, :d]
    s = tot[:, d : d + 1]
    safe = jnp.where(s == 0.0, 1.0, s)
    r = num / safe
    o_ref[...] = jnp.where(r > 0.0, r, jnp.exp(jnp.minimum(r, 0.0)) - 1.0)


def _make_sc_kernel(n, n_acc, da, e_pad, k):
    m = e_pad // NW
    n_chunks = m // k
    n_slice = n_acc // NS
    mesh = plsc.VectorSubcoreMesh(core_axis_name="c", subcore_axis_name="s",
                                  num_cores=NC, num_subcores=NS)
    cp = pltpu.CompilerParams()
    if "needs_layout_passes" in pltpu.CompilerParams.__dataclass_fields__:
        cp = dataclasses.replace(cp, needs_layout_passes=False)
    if "use_tc_tiling_on_sc" in pltpu.CompilerParams.__dataclass_fields__:
        cp = dataclasses.replace(cp, use_tc_tiling_on_sc=False)

    @functools.partial(
        pl.kernel,
        compiler_params=cp,
        out_type=jax.ShapeDtypeStruct((NC, n_acc, da), jnp.float32),
        mesh=mesh,
        scratch_types=[
            pltpu.VMEM((n,), jnp.float32),
            pltpu.VMEM((n,), jnp.float32),
            pltpu.VMEM((k,), jnp.int32),
            pltpu.VMEM((k,), jnp.int32),
            pltpu.VMEM((k,), jnp.float32),
            pltpu.VMEM((k, da), jnp.float32),
            pltpu.VMEM_SHARED((n_acc, da), jnp.float32),
            pltpu.SemaphoreType.DMA,
        ],
    )
    def sc_kernel(asrc_hbm, adst_hbm, src_hbm, dst_hbm, whaug_hbm, out_hbm,
                  asrc_v, adst_v, src_v, dst_v, ex_v, rows_v, acc, sem):
        cid = lax.axis_index("c")
        sid = lax.axis_index("s")
        wid = sid * NC + cid

        pltpu.sync_copy(asrc_hbm, asrc_v)
        pltpu.sync_copy(adst_hbm, adst_v)

        @pl.loop(0, k)
        def _(r):
            for c in range(da // L):
                rows_v[r, pl.ds(c * L, L)] = jnp.zeros((L,), jnp.float32)

        for p in range(n_slice // k):
            pltpu.sync_copy(rows_v, acc.at[pl.ds(sid * n_slice + p * k, k)])
        plsc.subcore_barrier()

        @pl.loop(0, n_chunks)
        def _(ci):
            base = wid * m + ci * k
            pltpu.sync_copy(src_hbm.at[pl.ds(base, k)], src_v)
            pltpu.sync_copy(dst_hbm.at[pl.ds(base, k)], dst_v)
            pltpu.async_copy(whaug_hbm.at[src_v], rows_v, sem).wait()

            @pl.loop(0, k // L)
            def _(g):
                s16 = src_v[pl.ds(g * L, L)]
                d16 = dst_v[pl.ds(g * L, L)]
                e = plsc.load_gather(asrc_v, [s16]) + plsc.load_gather(adst_v, [d16])
                e = jnp.maximum(e, 0.2 * e)
                ex_v[pl.ds(g * L, L)] = jnp.exp(e)

            @pl.loop(0, k)
            def _(r):
                b = plsc.load_gather(ex_v, [jnp.full((L,), r, jnp.int32)])
                for c in range(da // L):
                    rows_v[r, pl.ds(c * L, L)] = rows_v[r, pl.ds(c * L, L)] * b

            pltpu.sync_copy(rows_v, acc.at[dst_v], add=True)

        plsc.subcore_barrier()
        for p in range(n_slice // k):
            sl = pl.ds(sid * n_slice + p * k, k)
            pltpu.sync_copy(acc.at[sl], out_hbm.at[cid].at[sl])

    return sc_kernel


@jax.jit
def kernel(h, edge_index, W_fc, W_attn):
    b, n, d = h.shape
    e = edge_index.shape[1]
    da = d + 16
    n_acc = ((n + 1 + 1023) // 1024) * 1024
    k = 128
    e_pad = ((e + NW * k - 1) // (NW * k)) * (NW * k)

    h2 = h.reshape(n, d)
    a2 = W_attn.reshape(2, d).T
    src = edge_index[0]
    dst = edge_index[1]
    pad = e_pad - e
    if pad:
        src = jnp.concatenate([src, jnp.zeros((pad,), jnp.int32)])
        dst = jnp.concatenate([dst, jnp.full((pad,), n, jnp.int32)])

    br = 1000
    whaug, alphas = pl.pallas_call(
        _prep_body,
        grid=(n // br,),
        in_specs=[
            pl.BlockSpec((br, d), lambda i: (i, 0)),
            pl.BlockSpec((d, d), lambda i: (0, 0)),
            pl.BlockSpec((d, 2), lambda i: (0, 0)),
        ],
        out_specs=[
            pl.BlockSpec((br, da), lambda i: (i, 0)),
            pl.BlockSpec((br, 2), lambda i: (i, 0)),
        ],
        out_shape=[
            jax.ShapeDtypeStruct((n, da), jnp.float32),
            jax.ShapeDtypeStruct((n, 2), jnp.float32),
        ],
    )(h2, W_fc, a2)

    asrc = alphas[:, 0]
    adst = alphas[:, 1]

    parts = _make_sc_kernel(n, n_acc, da, e_pad, k)(asrc, adst, src, dst, whaug)

    bf = 1024
    out = pl.pallas_call(
        functools.partial(_fin_body, d=d),
        grid=(n_acc // bf,),
        in_specs=[pl.BlockSpec((NC, bf, da), lambda i: (0, i, 0))],
        out_specs=pl.BlockSpec((bf, d), lambda i: (i, 0)),
        out_shape=jax.ShapeDtypeStruct((n_acc, d), jnp.float32),
    )(parts)

    return out[:n].reshape(b, n, d)


# speedup vs baseline: 13.8603x; 1.2278x over previous
import dataclasses
import functools

import jax
import jax.numpy as jnp
from jax import lax
from jax.experimental import pallas as pl
from jax.experimental.pallas import tpu as pltpu
from jax.experimental.pallas import tpu_sc as plsc

NC, NS, L = 2, 16, 16
NW = NC * NS


def _prep_body(h_ref, wfc_ref, a2_ref, whaug_ref, alphas_ref):
    wh = lax.dot_general(h_ref[...], wfc_ref[...], (((1,), (1,)), ((), ())),
                         preferred_element_type=jnp.float32)
    alphas_ref[...] = jnp.dot(wh, a2_ref[...], preferred_element_type=jnp.float32)
    whaug_ref[:, : wh.shape[1]] = wh
    br = wh.shape[0]
    lane = lax.broadcasted_iota(jnp.int32, (br, 16), 1)
    whaug_ref[:, wh.shape[1]:] = jnp.where(lane == 0, 1.0, 0.0).astype(jnp.float32)


def _fin_body(p_ref, o_ref, *, d):
    tot = p_ref[0] + p_ref[1]
    num = tot[:, :d]
    s = tot[:, d : d + 1]
    safe = jnp.where(s == 0.0, 1.0, s)
    r = num / safe
    o_ref[...] = jnp.where(r > 0.0, r, jnp.exp(jnp.minimum(r, 0.0)) - 1.0)


def _sc_compiler_params():
    cp = pltpu.CompilerParams()
    if "needs_layout_passes" in pltpu.CompilerParams.__dataclass_fields__:
        cp = dataclasses.replace(cp, needs_layout_passes=False)
    if "use_tc_tiling_on_sc" in pltpu.CompilerParams.__dataclass_fields__:
        cp = dataclasses.replace(cp, use_tc_tiling_on_sc=False)
    return cp


def _sc_mesh():
    return plsc.VectorSubcoreMesh(core_axis_name="c", subcore_axis_name="s",
                                  num_cores=NC, num_subcores=NS)


def _make_score_kernel(n, e_pad, ks):
    m = e_pad // NW
    n_chunks = m // ks

    @functools.partial(
        pl.kernel,
        compiler_params=_sc_compiler_params(),
        out_type=jax.ShapeDtypeStruct((e_pad,), jnp.float32),
        mesh=_sc_mesh(),
        scratch_types=[
            pltpu.VMEM((n,), jnp.float32),
            pltpu.VMEM((n,), jnp.float32),
            pltpu.VMEM((ks,), jnp.int32),
            pltpu.VMEM((ks,), jnp.int32),
            pltpu.VMEM((ks,), jnp.int32),
            pltpu.VMEM((ks,), jnp.int32),
            pltpu.VMEM((ks,), jnp.float32),
            pltpu.VMEM((ks,), jnp.float32),
            pltpu.SemaphoreType.DMA,
            pltpu.SemaphoreType.DMA,
        ],
    )
    def score_kernel(asrc_hbm, adst_hbm, src_hbm, dst_hbm, ex_hbm,
                     asrc_v, adst_v, src_v0, src_v1, dst_v0, dst_v1,
                     ex_v0, ex_v1, sem0, sem1):
        cid = lax.axis_index("c")
        sid = lax.axis_index("s")
        wid = sid * NC + cid
        src_b = (src_v0, src_v1)
        dst_b = (dst_v0, dst_v1)
        ex_b = (ex_v0, ex_v1)
        sem_b = (sem0, sem1)

        pltpu.sync_copy(asrc_hbm, asrc_v)
        pltpu.sync_copy(adst_hbm, adst_v)

        def issue(ci, b):
            base = wid * m + ci * ks
            pltpu.async_copy(src_hbm.at[pl.ds(base, ks)], src_b[b], sem_b[b])
            pltpu.async_copy(dst_hbm.at[pl.ds(base, ks)], dst_b[b], sem_b[b])

        def process(ci, b):
            base = wid * m + ci * ks
            pltpu.make_async_copy(src_hbm.at[pl.ds(base, ks)], src_b[b], sem_b[b]).wait()
            pltpu.make_async_copy(dst_hbm.at[pl.ds(base, ks)], dst_b[b], sem_b[b]).wait()

            @pl.loop(0, ks // L, unroll=4)
            def _(g):
                s16 = src_b[b][pl.ds(g * L, L)]
                d16 = dst_b[b][pl.ds(g * L, L)]
                e = plsc.load_gather(asrc_v, [s16]) + plsc.load_gather(adst_v, [d16])
                e = jnp.maximum(e, 0.2 * e)
                ex_b[b][pl.ds(g * L, L)] = jnp.exp(e)

            pltpu.sync_copy(ex_b[b], ex_hbm.at[pl.ds(base, ks)])

        issue(0, 0)

        @pl.loop(0, n_chunks, step=2)
        def _(ci):
            issue(ci + 1, 1)
            process(ci, 0)

            @pl.when(ci + 2 < n_chunks)
            def _():
                issue(ci + 2, 0)

            process(ci + 1, 1)

    return score_kernel


def _make_scatter_kernel(n_acc, da, e_pad, k):
    m = e_pad // NW
    n_chunks = m // k
    n_slice = n_acc // NS

    @functools.partial(
        pl.kernel,
        compiler_params=_sc_compiler_params(),
        out_type=jax.ShapeDtypeStruct((NC, n_acc, da), jnp.float32),
        mesh=_sc_mesh(),
        scratch_types=[
            pltpu.VMEM((k,), jnp.int32),
            pltpu.VMEM((k,), jnp.int32),
            pltpu.VMEM((k,), jnp.int32),
            pltpu.VMEM((k,), jnp.int32),
            pltpu.VMEM((k,), jnp.float32),
            pltpu.VMEM((k,), jnp.float32),
            pltpu.VMEM((k, da), jnp.float32),
            pltpu.VMEM((k, da), jnp.float32),
            pltpu.VMEM_SHARED((n_acc, da), jnp.float32),
            pltpu.SemaphoreType.DMA,
            pltpu.SemaphoreType.DMA,
        ],
    )
    def scatter_kernel(src_hbm, dst_hbm, ex_hbm, whaug_hbm, out_hbm,
                       src_v0, src_v1, dst_v0, dst_v1, ex_v0, ex_v1,
                       rows_v0, rows_v1, acc, sem0, sem1):
        cid = lax.axis_index("c")
        sid = lax.axis_index("s")
        wid = sid * NC + cid
        src_b = (src_v0, src_v1)
        dst_b = (dst_v0, dst_v1)
        ex_b = (ex_v0, ex_v1)
        rows_b = (rows_v0, rows_v1)
        sem_b = (sem0, sem1)

        @pl.loop(0, k)
        def _(r):
            for c in range(da // L):
                rows_v0[r, pl.ds(c * L, L)] = jnp.zeros((L,), jnp.float32)

        for p in range(n_slice // k):
            pltpu.sync_copy(rows_v0, acc.at[pl.ds(sid * n_slice + p * k, k)])
        plsc.subcore_barrier()

        def issue(ci, b):
            base = wid * m + ci * k
            pltpu.sync_copy(src_hbm.at[pl.ds(base, k)], src_b[b])
            pltpu.async_copy(dst_hbm.at[pl.ds(base, k)], dst_b[b], sem_b[b])
            pltpu.async_copy(ex_hbm.at[pl.ds(base, k)], ex_b[b], sem_b[b])
            pltpu.async_copy(whaug_hbm.at[src_b[b]], rows_b[b], sem_b[b])

        def process(ci, b):
            base = wid * m + ci * k
            rows_v = rows_b[b]
            pltpu.make_async_copy(dst_hbm.at[pl.ds(base, k)], dst_b[b], sem_b[b]).wait()
            pltpu.make_async_copy(ex_hbm.at[pl.ds(base, k)], ex_b[b], sem_b[b]).wait()
            pltpu.make_async_copy(whaug_hbm.at[src_b[b]], rows_v, sem_b[b]).wait()

            @pl.loop(0, k, unroll=4)
            def _(r):
                bc = plsc.load_gather(ex_b[b], [jnp.full((L,), r, jnp.int32)])
                for c in range(da // L):
                    rows_v[r, pl.ds(c * L, L)] = rows_v[r, pl.ds(c * L, L)] * bc

            pltpu.sync_copy(rows_v, acc.at[dst_b[b]], add=True)

        issue(0, 0)

        @pl.loop(0, n_chunks, step=2)
        def _(ci):
            issue(ci + 1, 1)
            process(ci, 0)

            @pl.when(ci + 2 < n_chunks)
            def _():
                issue(ci + 2, 0)

            process(ci + 1, 1)

        plsc.subcore_barrier()
        for p in range(n_slice // k):
            sl = pl.ds(sid * n_slice + p * k, k)
            pltpu.sync_copy(acc.at[sl], out_hbm.at[cid].at[sl])

    return scatter_kernel


@jax.jit
def kernel(h, edge_index, W_fc, W_attn):
    b, n, d = h.shape
    e = edge_index.shape[1]
    da = d + 16
    n_acc = ((n + 1 + 1023) // 1024) * 1024
    k = 128
    step = NW * k * 2
    e_pad = ((e + step - 1) // step) * step

    h2 = h.reshape(n, d)
    a2 = W_attn.reshape(2, d).T
    src = edge_index[0]
    dst = edge_index[1]
    pad = e_pad - e
    if pad:
        src = jnp.concatenate([src, jnp.zeros((pad,), jnp.int32)])
        dst = jnp.concatenate([dst, jnp.full((pad,), n, jnp.int32)])

    br = 1000
    whaug, alphas = pl.pallas_call(
        _prep_body,
        grid=(n // br,),
        in_specs=[
            pl.BlockSpec((br, d), lambda i: (i, 0)),
            pl.BlockSpec((d, d), lambda i: (0, 0)),
            pl.BlockSpec((d, 2), lambda i: (0, 0)),
        ],
        out_specs=[
            pl.BlockSpec((br, da), lambda i: (i, 0)),
            pl.BlockSpec((br, 2), lambda i: (i, 0)),
        ],
        out_shape=[
            jax.ShapeDtypeStruct((n, da), jnp.float32),
            jax.ShapeDtypeStruct((n, 2), jnp.float32),
        ],
    )(h2, W_fc, a2)

    asrc = alphas[:, 0]
    adst = alphas[:, 1]

    ex = _make_score_kernel(n, e_pad, 512)(asrc, adst, src, dst)
    parts = _make_scatter_kernel(n_acc, da, e_pad, k)(src, dst, ex, whaug)

    bf = 1024
    out = pl.pallas_call(
        functools.partial(_fin_body, d=d),
        grid=(n_acc // bf,),
        in_specs=[pl.BlockSpec((NC, bf, da), lambda i: (0, i, 0))],
        out_specs=pl.BlockSpec((bf, d), lambda i: (i, 0)),
        out_shape=jax.ShapeDtypeStruct((n_acc, d), jnp.float32),
    )(parts)

    return out[:n].reshape(b, n, d)


# speedup vs baseline: 13.8737x; 1.0010x over previous
import dataclasses
import functools

import jax
import jax.numpy as jnp
from jax import lax
from jax.experimental import pallas as pl
from jax.experimental.pallas import tpu as pltpu
from jax.experimental.pallas import tpu_sc as plsc

NC, NS, L = 2, 16, 16
NW = NC * NS


def _prep_body(h_ref, wfc_ref, a2_ref, whaug_ref, alphas_ref):
    wh = lax.dot_general(h_ref[...], wfc_ref[...], (((1,), (1,)), ((), ())),
                         preferred_element_type=jnp.float32)
    alphas_ref[...] = jnp.dot(wh, a2_ref[...], preferred_element_type=jnp.float32)
    whaug_ref[:, : wh.shape[1]] = wh
    br = wh.shape[0]
    lane = lax.broadcasted_iota(jnp.int32, (br, 16), 1)
    whaug_ref[:, wh.shape[1]:] = jnp.where(lane == 0, 1.0, 0.0).astype(jnp.float32)


def _fin_body(p_ref, o_ref, *, d):
    tot = p_ref[0] + p_ref[1]
    num = tot[:, :d]
    s = tot[:, d : d + 1]
    safe = jnp.where(s == 0.0, 1.0, s)
    r = num / safe
    o_ref[...] = jnp.where(r > 0.0, r, jnp.exp(jnp.minimum(r, 0.0)) - 1.0)


def _sc_compiler_params():
    cp = pltpu.CompilerParams()
    if "needs_layout_passes" in pltpu.CompilerParams.__dataclass_fields__:
        cp = dataclasses.replace(cp, needs_layout_passes=False)
    if "use_tc_tiling_on_sc" in pltpu.CompilerParams.__dataclass_fields__:
        cp = dataclasses.replace(cp, use_tc_tiling_on_sc=False)
    return cp


def _sc_mesh():
    return plsc.VectorSubcoreMesh(core_axis_name="c", subcore_axis_name="s",
                                  num_cores=NC, num_subcores=NS)


def _make_score_kernel(n, e_pad, ks):
    m = e_pad // NW
    n_chunks = m // ks

    @functools.partial(
        pl.kernel,
        compiler_params=_sc_compiler_params(),
        out_type=jax.ShapeDtypeStruct((e_pad,), jnp.float32),
        mesh=_sc_mesh(),
        scratch_types=[
            pltpu.VMEM((n,), jnp.float32),
            pltpu.VMEM((n,), jnp.float32),
            pltpu.VMEM((ks,), jnp.int32),
            pltpu.VMEM((ks,), jnp.int32),
            pltpu.VMEM((ks,), jnp.int32),
            pltpu.VMEM((ks,), jnp.int32),
            pltpu.VMEM((ks,), jnp.float32),
            pltpu.VMEM((ks,), jnp.float32),
            pltpu.SemaphoreType.DMA,
            pltpu.SemaphoreType.DMA,
        ],
    )
    def score_kernel(asrc_hbm, adst_hbm, src_hbm, dst_hbm, ex_hbm,
                     asrc_v, adst_v, src_v0, src_v1, dst_v0, dst_v1,
                     ex_v0, ex_v1, sem0, sem1):
        cid = lax.axis_index("c")
        sid = lax.axis_index("s")
        wid = sid * NC + cid
        src_b = (src_v0, src_v1)
        dst_b = (dst_v0, dst_v1)
        ex_b = (ex_v0, ex_v1)
        sem_b = (sem0, sem1)

        pltpu.sync_copy(asrc_hbm, asrc_v)
        pltpu.sync_copy(adst_hbm, adst_v)

        def issue(ci, b):
            base = wid * m + ci * ks
            pltpu.async_copy(src_hbm.at[pl.ds(base, ks)], src_b[b], sem_b[b])
            pltpu.async_copy(dst_hbm.at[pl.ds(base, ks)], dst_b[b], sem_b[b])

        def process(ci, b):
            base = wid * m + ci * ks
            pltpu.make_async_copy(src_hbm.at[pl.ds(base, ks)], src_b[b], sem_b[b]).wait()
            pltpu.make_async_copy(dst_hbm.at[pl.ds(base, ks)], dst_b[b], sem_b[b]).wait()

            @pl.loop(0, ks // L, unroll=4)
            def _(g):
                s16 = src_b[b][pl.ds(g * L, L)]
                d16 = dst_b[b][pl.ds(g * L, L)]
                e = plsc.load_gather(asrc_v, [s16]) + plsc.load_gather(adst_v, [d16])
                e = jnp.maximum(e, 0.2 * e)
                ex_b[b][pl.ds(g * L, L)] = jnp.exp(e)

            pltpu.sync_copy(ex_b[b], ex_hbm.at[pl.ds(base, ks)])

        issue(0, 0)

        @pl.loop(0, n_chunks, step=2)
        def _(ci):
            issue(ci + 1, 1)
            process(ci, 0)

            @pl.when(ci + 2 < n_chunks)
            def _():
                issue(ci + 2, 0)

            process(ci + 1, 1)

    return score_kernel


def _make_scatter_kernel(n_acc, da, e_pad, k):
    m = e_pad // NW
    n_chunks = m // k
    n_slice = n_acc // NS

    @functools.partial(
        pl.kernel,
        compiler_params=_sc_compiler_params(),
        out_type=jax.ShapeDtypeStruct((NC, n_acc, da), jnp.float32),
        mesh=_sc_mesh(),
        scratch_types=[
            pltpu.VMEM((k,), jnp.int32),
            pltpu.VMEM((k,), jnp.int32),
            pltpu.VMEM((k,), jnp.int32),
            pltpu.VMEM((k,), jnp.int32),
            pltpu.VMEM((k,), jnp.float32),
            pltpu.VMEM((k,), jnp.float32),
            pltpu.VMEM((k, da), jnp.float32),
            pltpu.VMEM((k, da), jnp.float32),
            pltpu.VMEM_SHARED((n_acc, da), jnp.float32),
            pltpu.SemaphoreType.DMA,
            pltpu.SemaphoreType.DMA,
            pltpu.SemaphoreType.DMA,
            pltpu.SemaphoreType.DMA,
        ],
    )
    def scatter_kernel(src_hbm, dst_hbm, ex_hbm, whaug_hbm, out_hbm,
                       src_v0, src_v1, dst_v0, dst_v1, ex_v0, ex_v1,
                       rows_v0, rows_v1, acc, sem0, sem1, ssem0, ssem1):
        cid = lax.axis_index("c")
        sid = lax.axis_index("s")
        wid = sid * NC + cid
        src_b = (src_v0, src_v1)
        dst_b = (dst_v0, dst_v1)
        ex_b = (ex_v0, ex_v1)
        rows_b = (rows_v0, rows_v1)
        sem_b = (sem0, sem1)
        ssem_b = (ssem0, ssem1)

        @pl.loop(0, k)
        def _(r):
            for c in range(da // L):
                rows_v0[r, pl.ds(c * L, L)] = jnp.zeros((L,), jnp.float32)

        for p in range(n_slice // k):
            pltpu.sync_copy(rows_v0, acc.at[pl.ds(sid * n_slice + p * k, k)])
        plsc.subcore_barrier()

        def wait_scatter(b):
            pltpu.make_async_copy(rows_b[b], acc.at[dst_b[b]], ssem_b[b]).wait()

        def issue(ci, b, wait_prev):
            if wait_prev:
                @pl.when(ci >= 2)
                def _():
                    wait_scatter(b)
            base = wid * m + ci * k
            pltpu.sync_copy(src_hbm.at[pl.ds(base, k)], src_b[b])
            pltpu.async_copy(dst_hbm.at[pl.ds(base, k)], dst_b[b], sem_b[b])
            pltpu.async_copy(ex_hbm.at[pl.ds(base, k)], ex_b[b], sem_b[b])
            pltpu.async_copy(whaug_hbm.at[src_b[b]], rows_b[b], sem_b[b])

        def process(ci, b):
            base = wid * m + ci * k
            rows_v = rows_b[b]
            pltpu.make_async_copy(dst_hbm.at[pl.ds(base, k)], dst_b[b], sem_b[b]).wait()
            pltpu.make_async_copy(ex_hbm.at[pl.ds(base, k)], ex_b[b], sem_b[b]).wait()
            pltpu.make_async_copy(whaug_hbm.at[src_b[b]], rows_v, sem_b[b]).wait()

            @pl.loop(0, k, unroll=4)
            def _(r):
                bc = plsc.load_gather(ex_b[b], [jnp.full((L,), r, jnp.int32)])
                for c in range(da // L):
                    rows_v[r, pl.ds(c * L, L)] = rows_v[r, pl.ds(c * L, L)] * bc

            pltpu.async_copy(rows_v, acc.at[dst_b[b]], ssem_b[b], add=True)

        issue(0, 0, False)

        @pl.loop(0, n_chunks, step=2)
        def _(ci):
            issue(ci + 1, 1, True)
            process(ci, 0)

            @pl.when(ci + 2 < n_chunks)
            def _():
                issue(ci + 2, 0, True)

            process(ci + 1, 1)

        wait_scatter(0)
        wait_scatter(1)
        plsc.subcore_barrier()
        for p in range(n_slice // k):
            sl = pl.ds(sid * n_slice + p * k, k)
            pltpu.sync_copy(acc.at[sl], out_hbm.at[cid].at[sl])

    return scatter_kernel


@jax.jit
def kernel(h, edge_index, W_fc, W_attn):
    b, n, d = h.shape
    e = edge_index.shape[1]
    da = d + 16
    n_acc = ((n + 1 + 1023) // 1024) * 1024
    k = 128
    step = NW * k * 2
    e_pad = ((e + step - 1) // step) * step

    h2 = h.reshape(n, d)
    a2 = W_attn.reshape(2, d).T
    src = edge_index[0]
    dst = edge_index[1]
    pad = e_pad - e
    if pad:
        src = jnp.concatenate([src, jnp.zeros((pad,), jnp.int32)])
        dst = jnp.concatenate([dst, jnp.full((pad,), n, jnp.int32)])

    br = 1000
    whaug, alphas = pl.pallas_call(
        _prep_body,
        grid=(n // br,),
        in_specs=[
            pl.BlockSpec((br, d), lambda i: (i, 0)),
            pl.BlockSpec((d, d), lambda i: (0, 0)),
            pl.BlockSpec((d, 2), lambda i: (0, 0)),
        ],
        out_specs=[
            pl.BlockSpec((br, da), lambda i: (i, 0)),
            pl.BlockSpec((br, 2), lambda i: (i, 0)),
        ],
        out_shape=[
            jax.ShapeDtypeStruct((n, da), jnp.float32),
            jax.ShapeDtypeStruct((n, 2), jnp.float32),
        ],
    )(h2, W_fc, a2)

    asrc = alphas[:, 0]
    adst = alphas[:, 1]

    ex = _make_score_kernel(n, e_pad, 512)(asrc, adst, src, dst)
    parts = _make_scatter_kernel(n_acc, da, e_pad, k)(src, dst, ex, whaug)

    bf = 1024
    out = pl.pallas_call(
        functools.partial(_fin_body, d=d),
        grid=(n_acc // bf,),
        in_specs=[pl.BlockSpec((NC, bf, da), lambda i: (0, i, 0))],
        out_specs=pl.BlockSpec((bf, d), lambda i: (i, 0)),
        out_shape=jax.ShapeDtypeStruct((n_acc, d), jnp.float32),
    )(parts)

    return out[:n].reshape(b, n, d)
